# baseline (speedup 1.0000x reference)
import math

import jax
import jax.numpy as jnp
from jax.experimental import pallas as pl
from jax.experimental.pallas import tpu as pltpu

SCALE = 16
HL, WL = 14, 14
CF = 96
HH, WH = 224, 224
_TAPS = [(dy, dx) for dy in range(-2, 3) for dx in range(-2, 3)
         if dy * dy + dx * dx <= 4]
_NT = len(_TAPS)


def _up16(x):
  return jnp.repeat(x, SCALE, axis=1)


def _shift_up(xu, dx):
  s = SCALE * dx
  if dx == 0:
    return xu
  if dx > 0:
    return jnp.concatenate(
        [xu[:, s:]] + [xu[:, WH - SCALE:]] * dx, axis=1)
  return jnp.concatenate(
      [xu[:, :SCALE]] * (-dx) + [xu[:, :WH + s]], axis=1)


def _body(feat_ref, gstrip_ref, grow_ref, par_ref, out_ref, fup_ref, w_ref,
          fball_ref, gball_ref):
  u = pl.program_id(0)
  f32 = jnp.float32
  u_f = u.astype(f32)

  x_i = jax.lax.broadcasted_iota(jnp.int32, (1, WH), 1)
  x_f = x_i.astype(f32)
  jj = jax.lax.broadcasted_iota(jnp.int32, (WL, WH), 0)
  xx = jax.lax.broadcasted_iota(jnp.int32, (WL, WH), 1)
  m7 = (xx == jj * SCALE + 7).astype(f32)
  m8 = (xx == jj * SCALE + 8).astype(f32)

  @pl.when(u == 0)
  def _precompute():
    for i in range(HL):
      fball_ref[i] = _up16(feat_ref[i])
      r2 = grow_ref[:, pl.ds(i, 1), pl.ds(7, 2), :]
      row7 = r2[:, 0, 0, :][:, None, :]
      row8 = r2[:, 0, 1, :][:, None, :]
      v00 = jnp.sum(row7 * m7[None], axis=2)
      v01 = jnp.sum(row7 * m8[None], axis=2)
      v10 = jnp.sum(row8 * m7[None], axis=2)
      v11 = jnp.sum(row8 * m8[None], axis=2)
      gball_ref[i] = _up16(0.25 * (((v00 + v01) + v10) + v11))

  p = par_ref[:, pl.ds(u, 1), :, :].reshape(4, WL)
  sx = jnp.maximum(jnp.exp(p[0:1]), 1e-6)
  sy = jnp.maximum(jnp.exp(p[1:2]), 1e-6)
  th = math.pi * jnp.tanh(p[2:3])
  sr = jnp.maximum(jnp.exp(p[3:4]), 1e-6)
  D = jnp.concatenate([
      jnp.cos(th), jnp.sin(th),
      2.0 * sx ** 2 + 1e-8,
      2.0 * sy ** 2 + 1e-8,
      2.0 * sr ** 2 + 1e-8,
      jnp.clip(2.0 * jnp.maximum(sx, sy), 1.0, 2.0) ** 2,
  ], axis=0)
  Dup = _up16(D)
  cos_up, sin_up = Dup[0:1], Dup[1:2]
  d1_up, d2_up = Dup[2:3], Dup[3:4]
  d3_up, rsq_up = Dup[4:5], Dup[5:6]

  gs = gstrip_ref[...]

  riota = jax.lax.broadcasted_iota(jnp.int32, (SCALE, 1), 0).astype(f32)
  den = jnp.zeros((SCALE, WH), f32)

  ti = 0
  for dy in range(-2, 3):
    ui = jnp.clip(u + dy, 0, HL - 1)
    ui_f = ui.astype(f32)
    gbase = gball_ref[pl.ds(ui, 1), :, :].reshape(3, WH)
    fbase = fball_ref[pl.ds(ui, 1), :, :].reshape(CF, WH)
    cur_dy = (u_f - ui_f) + (riota - (SCALE - 1) / 2.0) / SCALE

    for dx in range(-2, 3):
      c2 = float(dy * dy + dx * dx)
      if c2 > 4.0:
        continue
      vi_x = jnp.clip(x_i // SCALE + dx, 0, WL - 1).astype(f32)
      cur_dx = (x_f - (vi_x * SCALE + (SCALE - 1) / 2.0)) / SCALE
      a = cur_dx * cos_up + cur_dy * sin_up
      b = (-cur_dx) * sin_up + cur_dy * cos_up
      logw = (-(a * a)) / d1_up - (b * b) / d2_up
      gup = _shift_up(gbase, dx)
      gd = ((gs[0] - gup[0:1]) ** 2 + (gs[1] - gup[1:2]) ** 2
            + (gs[2] - gup[2:3]) ** 2)
      logw = logw - gd / d3_up
      w = jnp.exp(logw)
      if c2 > 1.0:
        w = w * (c2 <= rsq_up).astype(f32)
      den = den + w
      w_ref[ti] = w
      fup_ref[ti] = _shift_up(fbase, dx)
      ti += 1
  assert ti == _NT

  invd = 1.0 / jnp.maximum(den, 1e-8)
  for r in range(0, SCALE, 2):
    acc0 = acc1 = None
    for ti in range(_NT):
      fv = fup_ref[ti]
      t0 = fv * w_ref[ti, r, :]
      t1 = fv * w_ref[ti, r + 1, :]
      acc0 = t0 if acc0 is None else acc0 + t0
      acc1 = t1 if acc1 is None else acc1 + t1
    out_ref[:, r, :] = acc0 * invd[r, :]
    out_ref[:, r + 1, :] = acc1 * invd[r + 1, :]


def kernel(feat_lr, guide_hr, sx_raw, sy_raw, th_raw, sr_raw):
  f32 = jnp.float32
  feat_t = jnp.transpose(feat_lr[0].astype(f32), (1, 0, 2))
  guide = guide_hr[0].astype(f32)
  guide4 = guide.reshape(3, HL, SCALE, WH)
  par = jnp.concatenate([
      sx_raw, sy_raw, th_raw, sr_raw], axis=1)[0].astype(f32)
  par = par.reshape(4, HL, 1, WL)

  out = pl.pallas_call(
      _body,
      grid=(HL,),
      in_specs=[
          pl.BlockSpec((HL, CF, WL), lambda u: (0, 0, 0)),
          pl.BlockSpec((3, SCALE, WH), lambda u: (0, u, 0)),
          pl.BlockSpec((3, HL, SCALE, WH), lambda u: (0, 0, 0, 0)),
          pl.BlockSpec((4, HL, 1, WL), lambda u: (0, 0, 0, 0)),
      ],
      out_specs=pl.BlockSpec((CF, SCALE, WH), lambda u: (0, u, 0)),
      out_shape=jax.ShapeDtypeStruct((CF, HH, WH), f32),
      scratch_shapes=[
          pltpu.VMEM((_NT, CF, WH), f32),
          pltpu.VMEM((_NT, SCALE, WH), f32),
          pltpu.VMEM((HL, CF, WH), f32),
          pltpu.VMEM((HL, 3, WH), f32),
      ],
  )(feat_t, guide, guide4, par)
  return out[None].astype(feat_lr.dtype)


# speedup vs baseline: 1.0053x; 1.0053x over previous
import math

import jax
import jax.numpy as jnp
from jax.experimental import pallas as pl
from jax.experimental.pallas import tpu as pltpu

SCALE = 16
HL, WL = 14, 14
CF = 96
HH, WH = 224, 224
_TAPS = [(dy, dx) for dy in range(-2, 3) for dx in range(-2, 3)
         if dy * dy + dx * dx <= 4]
_NT = len(_TAPS)


def _up16(x):
  return jnp.repeat(x, SCALE, axis=1)


def _shift_up(xu, dx):
  s = SCALE * dx
  if dx == 0:
    return xu
  if dx > 0:
    return jnp.concatenate(
        [xu[:, s:]] + [xu[:, WH - SCALE:]] * dx, axis=1)
  return jnp.concatenate(
      [xu[:, :SCALE]] * (-dx) + [xu[:, :WH + s]], axis=1)


def _body(feat_ref, gstrip_ref, grow_ref, par_ref, out_ref, fup_ref, w_ref,
          fball_ref, gball_ref):
  u = pl.program_id(0)
  f32 = jnp.float32
  u_f = u.astype(f32)

  x_i = jax.lax.broadcasted_iota(jnp.int32, (1, WH), 1)
  x_f = x_i.astype(f32)

  @pl.when(u == 0)
  def _precompute():
    jj = jax.lax.broadcasted_iota(jnp.int32, (WL, WH), 0)
    xx = jax.lax.broadcasted_iota(jnp.int32, (WL, WH), 1)
    m7 = (xx == jj * SCALE + 7).astype(f32)
    m8 = (xx == jj * SCALE + 8).astype(f32)
    for i in range(HL):
      fball_ref[i] = _up16(feat_ref[i])
      r2 = grow_ref[:, pl.ds(i, 1), pl.ds(7, 2), :]
      row7 = r2[:, 0, 0, :][:, None, :]
      row8 = r2[:, 0, 1, :][:, None, :]
      v00 = jnp.sum(row7 * m7[None], axis=2)
      v01 = jnp.sum(row7 * m8[None], axis=2)
      v10 = jnp.sum(row8 * m7[None], axis=2)
      v11 = jnp.sum(row8 * m8[None], axis=2)
      gball_ref[i] = _up16(0.25 * (((v00 + v01) + v10) + v11))

  p = par_ref[:, pl.ds(u, 1), :, :].reshape(4, WL)
  sx = jnp.maximum(jnp.exp(p[0:1]), 1e-6)
  sy = jnp.maximum(jnp.exp(p[1:2]), 1e-6)
  th = math.pi * jnp.tanh(p[2:3])
  sr = jnp.maximum(jnp.exp(p[3:4]), 1e-6)
  D = jnp.concatenate([
      jnp.cos(th), jnp.sin(th),
      2.0 * sx ** 2 + 1e-8,
      2.0 * sy ** 2 + 1e-8,
      2.0 * sr ** 2 + 1e-8,
      jnp.clip(2.0 * jnp.maximum(sx, sy), 1.0, 2.0) ** 2,
  ], axis=0)
  Dup = _up16(D)
  cos_up, sin_up = Dup[0:1], Dup[1:2]
  d1_up, d2_up = Dup[2:3], Dup[3:4]
  d3_up, rsq_up = Dup[4:5], Dup[5:6]

  gs = gstrip_ref[...]

  riota = jax.lax.broadcasted_iota(jnp.int32, (SCALE, 1), 0).astype(f32)
  den = jnp.zeros((SCALE, WH), f32)

  cdx_cos = {}
  cdx_sin = {}
  for dx in range(-2, 3):
    vi_x = jnp.clip(x_i // SCALE + dx, 0, WL - 1).astype(f32)
    cur_dx = (x_f - (vi_x * SCALE + (SCALE - 1) / 2.0)) / SCALE
    cdx_cos[dx] = cur_dx * cos_up
    cdx_sin[dx] = (-cur_dx) * sin_up

  ti = 0
  for dy in range(-2, 3):
    ui = jnp.clip(u + dy, 0, HL - 1)
    ui_f = ui.astype(f32)
    gbase = gball_ref[pl.ds(ui, 1), :, :].reshape(3, WH)
    fbase = fball_ref[pl.ds(ui, 1), :, :].reshape(CF, WH)
    cur_dy = (u_f - ui_f) + (riota - (SCALE - 1) / 2.0) / SCALE
    cdy_sin = cur_dy * sin_up
    cdy_cos = cur_dy * cos_up

    for dx in range(-2, 3):
      c2 = float(dy * dy + dx * dx)
      if c2 > 4.0:
        continue
      a = cdx_cos[dx] + cdy_sin
      b = cdx_sin[dx] + cdy_cos
      logw = (-(a * a)) / d1_up - (b * b) / d2_up
      gup = _shift_up(gbase, dx)
      gd = ((gs[0] - gup[0:1]) ** 2 + (gs[1] - gup[1:2]) ** 2
            + (gs[2] - gup[2:3]) ** 2)
      logw = logw - gd / d3_up
      w = jnp.exp(logw)
      if c2 > 1.0:
        w = w * (c2 <= rsq_up).astype(f32)
      den = den + w
      w_ref[ti] = w
      fup_ref[ti] = _shift_up(fbase, dx)
      ti += 1
  assert ti == _NT

  invd = 1.0 / jnp.maximum(den, 1e-8)
  for r in range(0, SCALE, 2):
    acc0 = acc1 = None
    for ti in range(_NT):
      fv = fup_ref[ti]
      t0 = fv * w_ref[ti, r, :]
      t1 = fv * w_ref[ti, r + 1, :]
      acc0 = t0 if acc0 is None else acc0 + t0
      acc1 = t1 if acc1 is None else acc1 + t1
    out_ref[:, r, :] = acc0 * invd[r, :]
    out_ref[:, r + 1, :] = acc1 * invd[r + 1, :]


def kernel(feat_lr, guide_hr, sx_raw, sy_raw, th_raw, sr_raw):
  f32 = jnp.float32
  feat_t = jnp.transpose(feat_lr[0].astype(f32), (1, 0, 2))
  guide = guide_hr[0].astype(f32)
  guide4 = guide.reshape(3, HL, SCALE, WH)
  par = jnp.concatenate([
      sx_raw, sy_raw, th_raw, sr_raw], axis=1)[0].astype(f32)
  par = par.reshape(4, HL, 1, WL)

  out = pl.pallas_call(
      _body,
      grid=(HL,),
      in_specs=[
          pl.BlockSpec((HL, CF, WL), lambda u: (0, 0, 0)),
          pl.BlockSpec((3, SCALE, WH), lambda u: (0, u, 0)),
          pl.BlockSpec((3, HL, SCALE, WH), lambda u: (0, 0, 0, 0)),
          pl.BlockSpec((4, HL, 1, WL), lambda u: (0, 0, 0, 0)),
      ],
      out_specs=pl.BlockSpec((CF, SCALE, WH), lambda u: (0, u, 0)),
      out_shape=jax.ShapeDtypeStruct((CF, HH, WH), f32),
      scratch_shapes=[
          pltpu.VMEM((_NT, CF, WH), f32),
          pltpu.VMEM((_NT, SCALE, WH), f32),
          pltpu.VMEM((HL, CF, WH), f32),
          pltpu.VMEM((HL, 3, WH), f32),
      ],
  )(feat_t, guide, guide4, par)
  return out[None].astype(feat_lr.dtype)


# speedup vs baseline: 1.0201x; 1.0147x over previous
import math

import jax
import jax.numpy as jnp
from jax.experimental import pallas as pl
from jax.experimental.pallas import tpu as pltpu

SCALE = 16
HL, WL = 14, 14
CF = 96
HH, WH = 224, 224
_TAPS = [(dy, dx) for dy in range(-2, 3) for dx in range(-2, 3)
         if dy * dy + dx * dx <= 4]
_NT = len(_TAPS)


def _up16(x):
  return jnp.repeat(x, SCALE, axis=1)


def _shift_up(xu, dx):
  s = SCALE * dx
  if dx == 0:
    return xu
  if dx > 0:
    return jnp.concatenate(
        [xu[:, s:]] + [xu[:, WH - SCALE:]] * dx, axis=1)
  return jnp.concatenate(
      [xu[:, :SCALE]] * (-dx) + [xu[:, :WH + s]], axis=1)


def _body(feat_ref, gstrip_ref, grow_ref, par_ref, out_ref, fup_ref, w_ref,
          fball_ref, gball_ref):
  u = pl.program_id(0)
  f32 = jnp.float32
  u_f = u.astype(f32)

  x_i = jax.lax.broadcasted_iota(jnp.int32, (1, WH), 1)
  x_f = x_i.astype(f32)

  @pl.when(u == 0)
  def _precompute():
    jj = jax.lax.broadcasted_iota(jnp.int32, (WL, WH), 0)
    xx = jax.lax.broadcasted_iota(jnp.int32, (WL, WH), 1)
    m7 = (xx == jj * SCALE + 7).astype(f32)
    m8 = (xx == jj * SCALE + 8).astype(f32)
    fall = feat_ref[...].reshape(HL * CF, WL)
    fball_ref[...] = _up16(fall).reshape(HL, CF, WH)
    for i in range(HL):
      r2 = grow_ref[:, pl.ds(i, 1), pl.ds(7, 2), :]
      row7 = r2[:, 0, 0, :][:, None, :]
      row8 = r2[:, 0, 1, :][:, None, :]
      v00 = jnp.sum(row7 * m7[None], axis=2)
      v01 = jnp.sum(row7 * m8[None], axis=2)
      v10 = jnp.sum(row8 * m7[None], axis=2)
      v11 = jnp.sum(row8 * m8[None], axis=2)
      gball_ref[i] = _up16(0.25 * (((v00 + v01) + v10) + v11))

  p = par_ref[:, pl.ds(u, 1), :, :].reshape(4, WL)
  sx = jnp.maximum(jnp.exp(p[0:1]), 1e-6)
  sy = jnp.maximum(jnp.exp(p[1:2]), 1e-6)
  th = math.pi * jnp.tanh(p[2:3])
  sr = jnp.maximum(jnp.exp(p[3:4]), 1e-6)
  D = jnp.concatenate([
      jnp.cos(th), jnp.sin(th),
      2.0 * sx ** 2 + 1e-8,
      2.0 * sy ** 2 + 1e-8,
      2.0 * sr ** 2 + 1e-8,
      jnp.clip(2.0 * jnp.maximum(sx, sy), 1.0, 2.0) ** 2,
  ], axis=0)
  Dup = _up16(D)
  cos_up, sin_up = Dup[0:1], Dup[1:2]
  d1_up, d2_up = Dup[2:3], Dup[3:4]
  d3_up, rsq_up = Dup[4:5], Dup[5:6]

  gs = gstrip_ref[...]

  riota = jax.lax.broadcasted_iota(jnp.int32, (SCALE, 1), 0).astype(f32)
  den = jnp.zeros((SCALE, WH), f32)

  cdx_cos = {}
  cdx_sin = {}
  for dx in range(-2, 3):
    vi_x = jnp.clip(x_i // SCALE + dx, 0, WL - 1).astype(f32)
    cur_dx = (x_f - (vi_x * SCALE + (SCALE - 1) / 2.0)) / SCALE
    cdx_cos[dx] = cur_dx * cos_up
    cdx_sin[dx] = (-cur_dx) * sin_up

  ti = 0
  for dy in range(-2, 3):
    ui = jnp.clip(u + dy, 0, HL - 1)
    ui_f = ui.astype(f32)
    gbase = gball_ref[pl.ds(ui, 1), :, :].reshape(3, WH)
    fbase = fball_ref[pl.ds(ui, 1), :, :].reshape(CF, WH)
    cur_dy = (u_f - ui_f) + (riota - (SCALE - 1) / 2.0) / SCALE
    cdy_sin = cur_dy * sin_up
    cdy_cos = cur_dy * cos_up

    for dx in range(-2, 3):
      c2 = float(dy * dy + dx * dx)
      if c2 > 4.0:
        continue
      a = cdx_cos[dx] + cdy_sin
      b = cdx_sin[dx] + cdy_cos
      logw = (-(a * a)) / d1_up - (b * b) / d2_up
      gup = _shift_up(gbase, dx)
      gd = ((gs[0] - gup[0:1]) ** 2 + (gs[1] - gup[1:2]) ** 2
            + (gs[2] - gup[2:3]) ** 2)
      logw = logw - gd / d3_up
      w = jnp.exp(logw)
      if c2 > 1.0:
        w = w * (c2 <= rsq_up).astype(f32)
      den = den + w
      w_ref[ti] = w
      fup_ref[ti] = _shift_up(fbase, dx)
      ti += 1
  assert ti == _NT

  invd = 1.0 / jnp.maximum(den, 1e-8)
  for r in range(0, SCALE, 2):
    acc0 = acc1 = None
    for ti in range(_NT):
      fv = fup_ref[ti]
      t0 = fv * w_ref[ti, r, :]
      t1 = fv * w_ref[ti, r + 1, :]
      acc0 = t0 if acc0 is None else acc0 + t0
      acc1 = t1 if acc1 is None else acc1 + t1
    out_ref[:, r, :] = acc0 * invd[r, :]
    out_ref[:, r + 1, :] = acc1 * invd[r + 1, :]


def kernel(feat_lr, guide_hr, sx_raw, sy_raw, th_raw, sr_raw):
  f32 = jnp.float32
  feat_t = jnp.transpose(feat_lr[0].astype(f32), (1, 0, 2))
  guide = guide_hr[0].astype(f32)
  guide4 = guide.reshape(3, HL, SCALE, WH)
  par = jnp.concatenate([
      sx_raw, sy_raw, th_raw, sr_raw], axis=1)[0].astype(f32)
  par = par.reshape(4, HL, 1, WL)

  out = pl.pallas_call(
      _body,
      grid=(HL,),
      in_specs=[
          pl.BlockSpec((HL, CF, WL), lambda u: (0, 0, 0)),
          pl.BlockSpec((3, SCALE, WH), lambda u: (0, u, 0)),
          pl.BlockSpec((3, HL, SCALE, WH), lambda u: (0, 0, 0, 0)),
          pl.BlockSpec((4, HL, 1, WL), lambda u: (0, 0, 0, 0)),
      ],
      out_specs=pl.BlockSpec((CF, SCALE, WH), lambda u: (0, u, 0)),
      out_shape=jax.ShapeDtypeStruct((CF, HH, WH), f32),
      scratch_shapes=[
          pltpu.VMEM((_NT, CF, WH), f32),
          pltpu.VMEM((_NT, SCALE, WH), f32),
          pltpu.VMEM((HL, CF, WH), f32),
          pltpu.VMEM((HL, 3, WH), f32),
      ],
  )(feat_t, guide, guide4, par)
  return out[None].astype(feat_lr.dtype)


# speedup vs baseline: 1.0389x; 1.0184x over previous
import math

import jax
import jax.numpy as jnp
from jax.experimental import pallas as pl
from jax.experimental.pallas import tpu as pltpu

SCALE = 16
HL, WL = 14, 14
CF = 96
HH, WH = 224, 224
_TAPS = [(dy, dx) for dy in range(-2, 3) for dx in range(-2, 3)
         if dy * dy + dx * dx <= 4]
_NT = len(_TAPS)


def _up16(x):
  return jnp.repeat(x, SCALE, axis=1)


def _shift_up(xu, dx):
  s = SCALE * dx
  if dx == 0:
    return xu
  if dx > 0:
    return jnp.concatenate(
        [xu[:, s:]] + [xu[:, WH - SCALE:]] * dx, axis=1)
  return jnp.concatenate(
      [xu[:, :SCALE]] * (-dx) + [xu[:, :WH + s]], axis=1)


def _body(feat_ref, gstrip_ref, grow_ref, par_ref, out_ref, fup_ref, w_ref,
          fball_ref, gball_ref):
  u = pl.program_id(0)
  f32 = jnp.float32
  u_f = u.astype(f32)

  x_i = jax.lax.broadcasted_iota(jnp.int32, (1, WH), 1)
  x_f = x_i.astype(f32)

  @pl.when(u == 0)
  def _precompute():
    jj = jax.lax.broadcasted_iota(jnp.int32, (WL, WH), 0)
    xx = jax.lax.broadcasted_iota(jnp.int32, (WL, WH), 1)
    m7 = (xx == jj * SCALE + 7).astype(f32)
    m8 = (xx == jj * SCALE + 8).astype(f32)
    fall = feat_ref[...].reshape(HL * CF, WL)
    fball_ref[...] = _up16(fall).reshape(HL, CF, WH)
    glr_rows = []
    for i in range(HL):
      r2 = grow_ref[:, pl.ds(i, 1), pl.ds(7, 2), :]
      row7 = r2[:, 0, 0, :][:, None, :]
      row8 = r2[:, 0, 1, :][:, None, :]
      v00 = jnp.sum(row7 * m7[None], axis=2)
      v01 = jnp.sum(row7 * m8[None], axis=2)
      v10 = jnp.sum(row8 * m7[None], axis=2)
      v11 = jnp.sum(row8 * m8[None], axis=2)
      glr_rows.append(0.25 * (((v00 + v01) + v10) + v11))
    gall = jnp.concatenate(glr_rows, axis=0)
    gball_ref[...] = _up16(gall).reshape(HL, 3, WH)

  p = par_ref[:, pl.ds(u, 1), :, :].reshape(4, WL)
  sx = jnp.maximum(jnp.exp(p[0:1]), 1e-6)
  sy = jnp.maximum(jnp.exp(p[1:2]), 1e-6)
  th = math.pi * jnp.tanh(p[2:3])
  sr = jnp.maximum(jnp.exp(p[3:4]), 1e-6)
  D = jnp.concatenate([
      jnp.cos(th), jnp.sin(th),
      2.0 * sx ** 2 + 1e-8,
      2.0 * sy ** 2 + 1e-8,
      2.0 * sr ** 2 + 1e-8,
      jnp.clip(2.0 * jnp.maximum(sx, sy), 1.0, 2.0) ** 2,
  ], axis=0)
  Dup = _up16(D)
  cos_up, sin_up = Dup[0:1], Dup[1:2]
  d1_up, d2_up = Dup[2:3], Dup[3:4]
  d3_up, rsq_up = Dup[4:5], Dup[5:6]

  gs = gstrip_ref[...]

  riota = jax.lax.broadcasted_iota(jnp.int32, (SCALE, 1), 0).astype(f32)
  den = jnp.zeros((SCALE, WH), f32)

  cdx_cos = {}
  cdx_sin = {}
  for dx in range(-2, 3):
    vi_x = jnp.clip(x_i // SCALE + dx, 0, WL - 1).astype(f32)
    cur_dx = (x_f - (vi_x * SCALE + (SCALE - 1) / 2.0)) / SCALE
    cdx_cos[dx] = cur_dx * cos_up
    cdx_sin[dx] = (-cur_dx) * sin_up

  ti = 0
  for dy in range(-2, 3):
    ui = jnp.clip(u + dy, 0, HL - 1)
    ui_f = ui.astype(f32)
    gbase = gball_ref[pl.ds(ui, 1), :, :].reshape(3, WH)
    fbase = fball_ref[pl.ds(ui, 1), :, :].reshape(CF, WH)
    cur_dy = (u_f - ui_f) + (riota - (SCALE - 1) / 2.0) / SCALE
    cdy_sin = cur_dy * sin_up
    cdy_cos = cur_dy * cos_up

    for dx in range(-2, 3):
      c2 = float(dy * dy + dx * dx)
      if c2 > 4.0:
        continue
      a = cdx_cos[dx] + cdy_sin
      b = cdx_sin[dx] + cdy_cos
      logw = (-(a * a)) / d1_up - (b * b) / d2_up
      gup = _shift_up(gbase, dx)
      gd = ((gs[0] - gup[0:1]) ** 2 + (gs[1] - gup[1:2]) ** 2
            + (gs[2] - gup[2:3]) ** 2)
      logw = logw - gd / d3_up
      w = jnp.exp(logw)
      if c2 > 1.0:
        w = w * (c2 <= rsq_up).astype(f32)
      den = den + w
      w_ref[ti] = w
      fup_ref[ti] = _shift_up(fbase, dx)
      ti += 1
  assert ti == _NT

  invd = 1.0 / jnp.maximum(den, 1e-8)
  for r in range(0, SCALE, 2):
    acc0 = acc1 = None
    for ti in range(_NT):
      fv = fup_ref[ti]
      t0 = fv * w_ref[ti, r, :]
      t1 = fv * w_ref[ti, r + 1, :]
      acc0 = t0 if acc0 is None else acc0 + t0
      acc1 = t1 if acc1 is None else acc1 + t1
    out_ref[:, r, :] = acc0 * invd[r, :]
    out_ref[:, r + 1, :] = acc1 * invd[r + 1, :]


def kernel(feat_lr, guide_hr, sx_raw, sy_raw, th_raw, sr_raw):
  f32 = jnp.float32
  feat_t = jnp.transpose(feat_lr[0].astype(f32), (1, 0, 2))
  guide = guide_hr[0].astype(f32)
  guide4 = guide.reshape(3, HL, SCALE, WH)
  par = jnp.concatenate([
      sx_raw, sy_raw, th_raw, sr_raw], axis=1)[0].astype(f32)
  par = par.reshape(4, HL, 1, WL)

  out = pl.pallas_call(
      _body,
      grid=(HL,),
      in_specs=[
          pl.BlockSpec((HL, CF, WL), lambda u: (0, 0, 0)),
          pl.BlockSpec((3, SCALE, WH), lambda u: (0, u, 0)),
          pl.BlockSpec((3, HL, SCALE, WH), lambda u: (0, 0, 0, 0)),
          pl.BlockSpec((4, HL, 1, WL), lambda u: (0, 0, 0, 0)),
      ],
      out_specs=pl.BlockSpec((CF, SCALE, WH), lambda u: (0, u, 0)),
      out_shape=jax.ShapeDtypeStruct((CF, HH, WH), f32),
      scratch_shapes=[
          pltpu.VMEM((_NT, CF, WH), f32),
          pltpu.VMEM((_NT, SCALE, WH), f32),
          pltpu.VMEM((HL, CF, WH), f32),
          pltpu.VMEM((HL, 3, WH), f32),
      ],
  )(feat_t, guide, guide4, par)
  return out[None].astype(feat_lr.dtype)
